# baseline (speedup 1.0000x reference)
import functools

import jax
import jax.numpy as jnp
from jax import lax
from jax.experimental import pallas as pl
from jax.experimental.pallas import tpu as pltpu
from jax.experimental.pallas import tpu_sc as plsc

N_CODES = 1024
E_DIM = 64
N_TOK = 9216
BETA = 0.25
BPB = 8
R = BPB * 576
NB = N_TOK // R

_SC_INFO = plsc.get_sparse_core_info()
_NC, _NS = _SC_INFO.num_cores, _SC_INFO.num_subcores
_NW = _NC * _NS
_CHUNK = 96
_TPW = N_TOK // _NW
_CPW = _TPW // _CHUNK


def _tc_body(z_ref, emb_ref, idx_ref, loss_ref, plex_ref,
             cnt_acc, loss_acc):
    i = pl.program_id(0)
    z = z_ref[...].reshape(R, E_DIM)
    emb = emb_ref[...]

    sq = z * z
    zs = lax.dot_general(jnp.ones((1, E_DIM), jnp.float32), sq,
                         (((1,), (1,)), ((), ())),
                         preferred_element_type=jnp.float32,
                         precision=lax.Precision.HIGHEST)
    es = jnp.sum(emb * emb, axis=1, keepdims=True)
    mm = lax.dot_general(emb, z, (((1,), (1,)), ((), ())),
                         preferred_element_type=jnp.float32)
    d = (zs + es) - 2.0 * mm

    m = jnp.min(d, axis=0, keepdims=True)
    kiota = lax.broadcasted_iota(
        jnp.int32, (N_CODES, 1), 0).astype(jnp.float32)
    idxf = jnp.min(jnp.where(d == m, kiota, float(2 * N_CODES)),
                   axis=0, keepdims=True)
    idx_ref[pl.ds(i * R, R)] = idxf[0].astype(jnp.int32)

    onehot = (idxf == kiota).astype(jnp.float32)
    cnt = lax.dot_general(onehot, jnp.ones((1, R), jnp.float32),
                          (((1,), (1,)), ((), ())),
                          preferred_element_type=jnp.float32)
    bs = jnp.sum(m)

    @pl.when(i == 0)
    def _():
        cnt_acc[...] = cnt
        loss_acc[0, 0] = bs

    @pl.when(i > 0)
    def _():
        cnt_acc[...] = cnt_acc[...] + cnt
        loss_acc[0, 0] = loss_acc[0, 0] + bs

    @pl.when(i == NB - 1)
    def _():
        total = loss_acc[0, 0]
        loss_ref[...] = jnp.full(
            (1, 1), (1.0 + BETA) * (total / float(N_TOK * E_DIM)),
            dtype=jnp.float32)
        e_mean = cnt_acc[...] / float(N_TOK)
        ent = jnp.sum(e_mean * jnp.log(e_mean + 1e-10),
                      axis=0, keepdims=True)
        plex_ref[...] = jnp.exp(-ent)


def _tc_call(z, emb):
    return pl.pallas_call(
        _tc_body,
        grid=(NB,),
        in_specs=[
            pl.BlockSpec((BPB, 576, E_DIM), lambda i: (i, 0, 0)),
            pl.BlockSpec((N_CODES, E_DIM), lambda i: (0, 0)),
        ],
        out_specs=[
            pl.BlockSpec((N_TOK,), lambda i: (0,)),
            pl.BlockSpec((1, 1), lambda i: (0, 0)),
            pl.BlockSpec((1, 1), lambda i: (0, 0)),
        ],
        out_shape=[
            jax.ShapeDtypeStruct((N_TOK,), jnp.int32),
            jax.ShapeDtypeStruct((1, 1), jnp.float32),
            jax.ShapeDtypeStruct((1, 1), jnp.float32),
        ],
        scratch_shapes=[
            pltpu.VMEM((N_CODES, 1), jnp.float32),
            pltpu.SMEM((1, 1), jnp.float32),
        ],
    )(z, emb)


@functools.partial(
    pl.kernel,
    out_type=jax.ShapeDtypeStruct((16, 576, E_DIM), jnp.float32),
    mesh=plsc.VectorSubcoreMesh(core_axis_name="c", subcore_axis_name="s"),
    compiler_params=pltpu.CompilerParams(use_tc_tiling_on_sc=False),
    scratch_types=[
        pltpu.VMEM((_TPW,), jnp.int32),
        pltpu.VMEM((_TPW, E_DIM), jnp.float32),
        pltpu.SemaphoreType.DMA,
    ],
)
def _sc_gather(emb_hbm, idx_hbm, out_hbm, idx_v, rows_v, sem):
    wid = lax.axis_index("s") * _NC + lax.axis_index("c")
    base = wid * _TPW
    pltpu.sync_copy(idx_hbm.at[pl.ds(base, _TPW)], idx_v)
    copies = [
        pltpu.async_copy(
            emb_hbm.at[idx_v.at[pl.ds(j * _CHUNK, _CHUNK)]],
            rows_v.at[pl.ds(j * _CHUNK, _CHUNK)], sem)
        for j in range(_CPW)
    ]
    for c in copies:
        c.wait()
    b, half = wid // 2, wid % 2
    pltpu.sync_copy(rows_v, out_hbm.at[b, pl.ds(half * _TPW, _TPW)])


@jax.jit
def kernel(z, emb):
    idx, loss, plex = _tc_call(z, emb)
    z_q_st = _sc_gather(emb, idx)
    return (loss[0, 0], z_q_st, idx, plex[0, 0])


# speedup vs baseline: 1.0786x; 1.0786x over previous
import functools

import jax
import jax.numpy as jnp
from jax import lax
from jax.experimental import pallas as pl
from jax.experimental.pallas import tpu as pltpu
from jax.experimental.pallas import tpu_sc as plsc

N_CODES = 1024
E_DIM = 64
N_TOK = 9216
BETA = 0.25
BPB = 4
R = BPB * 576
NB = N_TOK // R

_SC_INFO = plsc.get_sparse_core_info()
_NC, _NS = _SC_INFO.num_cores, _SC_INFO.num_subcores
_NW = _NC * _NS
_CHUNK = 96
_TPW = N_TOK // _NW
_CPW = _TPW // _CHUNK


def _tc_a_body(z_ref, emb_ref, idx_ref, loss_ref, loss_acc):
    i = pl.program_id(0)
    z = z_ref[...].reshape(R, E_DIM)
    emb = emb_ref[...]

    sq = z * z
    zs = lax.dot_general(jnp.ones((1, E_DIM), jnp.float32), sq,
                         (((1,), (1,)), ((), ())),
                         preferred_element_type=jnp.float32,
                         precision=lax.Precision.HIGHEST)
    es = jnp.sum(emb * emb, axis=1, keepdims=True)
    mm = lax.dot_general(emb, z, (((1,), (1,)), ((), ())),
                         preferred_element_type=jnp.float32)
    d = (zs + es) - 2.0 * mm

    m = jnp.min(d, axis=0, keepdims=True)
    kiota = lax.broadcasted_iota(
        jnp.int32, (N_CODES, 1), 0).astype(jnp.float32)
    idxf = jnp.min(jnp.where(d == m, kiota, float(2 * N_CODES)),
                   axis=0, keepdims=True)
    idx_ref[pl.ds(i * R, R)] = idxf[0].astype(jnp.int32)

    bs = jnp.sum(m)

    @pl.when(i == 0)
    def _():
        loss_acc[0, 0] = bs

    @pl.when(i > 0)
    def _():
        loss_acc[0, 0] = loss_acc[0, 0] + bs

    @pl.when(i == NB - 1)
    def _():
        total = loss_acc[0, 0]
        loss_ref[...] = jnp.full(
            (1, 1), (1.0 + BETA) * (total / float(N_TOK * E_DIM)),
            dtype=jnp.float32)


def _tc_a(z, emb):
    return pl.pallas_call(
        _tc_a_body,
        grid=(NB,),
        in_specs=[
            pl.BlockSpec((BPB, 576, E_DIM), lambda i: (i, 0, 0)),
            pl.BlockSpec((N_CODES, E_DIM), lambda i: (0, 0)),
        ],
        out_specs=[
            pl.BlockSpec((N_TOK,), lambda i: (0,)),
            pl.BlockSpec((1, 1), lambda i: (0, 0)),
        ],
        out_shape=[
            jax.ShapeDtypeStruct((N_TOK,), jnp.int32),
            jax.ShapeDtypeStruct((1, 1), jnp.float32),
        ],
        scratch_shapes=[
            pltpu.SMEM((1, 1), jnp.float32),
        ],
    )(z, emb)


def _tc_b_body(idx_ref, plex_ref):
    idx = idx_ref[...].reshape(1, N_TOK).astype(jnp.float32)
    kiota = lax.broadcasted_iota(
        jnp.int32, (N_CODES, 1), 0).astype(jnp.float32)
    onehot = (idx == kiota).astype(jnp.float32)
    cnt = lax.dot_general(onehot, jnp.ones((1, N_TOK), jnp.float32),
                          (((1,), (1,)), ((), ())),
                          preferred_element_type=jnp.float32)
    e_mean = cnt / float(N_TOK)
    ent = jnp.sum(e_mean * jnp.log(e_mean + 1e-10),
                  axis=0, keepdims=True)
    plex_ref[...] = jnp.exp(-ent)


def _tc_b(idx):
    return pl.pallas_call(
        _tc_b_body,
        out_shape=jax.ShapeDtypeStruct((1, 1), jnp.float32),
    )(idx)


@functools.partial(
    pl.kernel,
    out_type=jax.ShapeDtypeStruct((16, 576, E_DIM), jnp.float32),
    mesh=plsc.VectorSubcoreMesh(core_axis_name="c", subcore_axis_name="s"),
    compiler_params=pltpu.CompilerParams(use_tc_tiling_on_sc=False),
    scratch_types=[
        pltpu.VMEM((_TPW,), jnp.int32),
        pltpu.VMEM((_TPW, E_DIM), jnp.float32),
        pltpu.SemaphoreType.DMA,
    ],
)
def _sc_gather(emb_hbm, idx_hbm, out_hbm, idx_v, rows_v, sem):
    wid = lax.axis_index("s") * _NC + lax.axis_index("c")
    base = wid * _TPW
    pltpu.sync_copy(idx_hbm.at[pl.ds(base, _TPW)], idx_v)
    copies = [
        pltpu.async_copy(
            emb_hbm.at[idx_v.at[pl.ds(j * _CHUNK, _CHUNK)]],
            rows_v.at[pl.ds(j * _CHUNK, _CHUNK)], sem)
        for j in range(_CPW)
    ]
    for c in copies:
        c.wait()
    b, half = wid // 2, wid % 2
    pltpu.sync_copy(rows_v, out_hbm.at[b, pl.ds(half * _TPW, _TPW)])


@jax.jit
def kernel(z, emb):
    idx, loss = _tc_a(z, emb)
    z_q_st = _sc_gather(emb, idx)
    plex = _tc_b(idx)
    return (loss[0, 0], z_q_st, idx, plex[0, 0])
